# baseline (speedup 1.0000x reference)
import functools

import jax
import jax.numpy as jnp
from jax import lax
from jax.experimental import pallas as pl
from jax.experimental.pallas import tpu as pltpu, tpu_sc as plsc

N_NODES = 100000
N_EDGES = 6400000
NC = 2
NS = 16
NW = NC * NS
CH = 128
N_CHUNKS = N_EDGES // CH
BASE_CHUNKS = N_CHUNKS // NW
REM_CHUNKS = N_CHUNKS % NW
N_PAD = 100096
ROWS_PER_TILE = N_PAD // NS


RING = 4


def _make_agg_generic(dp, split, kblk):
    mesh = plsc.VectorSubcoreMesh(
        core_axis_name="c", subcore_axis_name="s", num_cores=NC, num_subcores=NS
    )
    stride = NS if split else NW
    nb_total = N_CHUNKS // kblk
    assert nb_total * kblk == N_CHUNKS
    base_blocks = nb_total // stride
    rem_blocks = nb_total % stride
    table_shape = (
        jax.ShapeDtypeStruct((NC, N_NODES, dp), jnp.float32)
        if split else jax.ShapeDtypeStruct((N_NODES, dp), jnp.float32)
    )
    del table_shape

    @functools.partial(
        pl.kernel,
        out_type=jax.ShapeDtypeStruct((NC, N_PAD, dp), jnp.float32),
        mesh=mesh,
        scratch_types=[
            pltpu.VMEM((RING, kblk, CH), jnp.int32),
            pltpu.VMEM((RING, kblk, CH), jnp.int32),
            pltpu.VMEM((RING, kblk, CH, dp), jnp.float32),
            pltpu.VMEM_SHARED((N_PAD, dp), jnp.float32),
            pltpu.SemaphoreType.DMA((RING,)),
            pltpu.SemaphoreType.DMA((RING,)),
            pltpu.SemaphoreType.DMA((RING,)),
        ],
        compiler_params=pltpu.CompilerParams(use_tc_tiling_on_sc=False),
    )
    def agg_kernel(table, edges, zeros, out, src_v, dst_v, rows_v, acc,
                   sem_i, sem_g, sem_s):
        cid = lax.axis_index("c")
        sid = lax.axis_index("s")
        wid = sid if split else sid * NC + cid

        r0 = sid * ROWS_PER_TILE
        pltpu.sync_copy(
            zeros.at[pl.ds(r0, ROWS_PER_TILE)], acc.at[pl.ds(r0, ROWS_PER_TILE)]
        )
        plsc.subcore_barrier()

        n_blocks = jnp.where(wid < rem_blocks, base_blocks + 1, base_blocks)

        def tbl(k_idx):
            return (table.at[cid] if split else table).at[k_idx]

        def fire_idx(m):
            sl = jnp.bitwise_and(m, RING - 1)
            c0 = (wid + m * stride) * kblk
            pltpu.async_copy(edges.at[0, pl.ds(c0, kblk)], src_v.at[sl],
                             sem_i.at[sl])
            pltpu.async_copy(edges.at[1, pl.ds(c0, kblk)], dst_v.at[sl],
                             sem_i.at[sl])

        def drain_idx(m):
            sl = jnp.bitwise_and(m, RING - 1)
            c0 = (wid + m * stride) * kblk
            pltpu.make_async_copy(edges.at[0, pl.ds(c0, kblk)], src_v.at[sl],
                                  sem_i.at[sl]).wait()
            pltpu.make_async_copy(edges.at[1, pl.ds(c0, kblk)], dst_v.at[sl],
                                  sem_i.at[sl]).wait()

        def fire_gath(m):
            sl = jnp.bitwise_and(m, RING - 1)
            for k in range(kblk):
                pltpu.async_copy(tbl(src_v.at[sl, k]), rows_v.at[sl, k],
                                 sem_g.at[sl])

        def drain_gath(m):
            sl = jnp.bitwise_and(m, RING - 1)
            for k in range(kblk):
                pltpu.make_async_copy(tbl(src_v.at[sl, k]), rows_v.at[sl, k],
                                      sem_g.at[sl]).wait()

        def fire_scat(m):
            sl = jnp.bitwise_and(m, RING - 1)
            for k in range(kblk):
                pltpu.async_copy(rows_v.at[sl, k], acc.at[dst_v.at[sl, k]],
                                 sem_s.at[sl], add=True)

        def drain_scat(m):
            sl = jnp.bitwise_and(m, RING - 1)
            for k in range(kblk):
                pltpu.make_async_copy(rows_v.at[sl, k], acc.at[dst_v.at[sl, k]],
                                      sem_s.at[sl]).wait()

        fire_idx(0)
        fire_idx(1)
        drain_idx(0)
        fire_gath(0)

        @pl.loop(0, n_blocks)
        def _(m):
            drain_gath(m)
            fire_scat(m)

            @pl.when(m >= 2)
            def _():
                drain_scat(m - 2)

            @pl.when(m + 1 < n_blocks)
            def _():
                drain_idx(m + 1)
                fire_gath(m + 1)

            @pl.when(m + 2 < n_blocks)
            def _():
                fire_idx(m + 2)

        drain_scat(n_blocks - 2)
        drain_scat(n_blocks - 1)

        plsc.subcore_barrier()
        pltpu.sync_copy(
            acc.at[pl.ds(r0, ROWS_PER_TILE)], out.at[cid, pl.ds(r0, ROWS_PER_TILE)]
        )

    return agg_kernel


def _make_agg(dp, kblk):
    return _make_agg_generic(dp, split=False, kblk=kblk)


def _make_agg3():
    return _make_agg_generic(16, split=True, kblk=2)


_agg8 = _make_agg(8, kblk=8)
_agg16 = _make_agg(16, kblk=2)
_agg3 = _make_agg3()

_BLK = 2000
_GRID = N_NODES // _BLK


def _l1_body(p_ref, x_ref, wl_ref, bl_ref, wr_ref, h_ref, inv_ref):
    p = p_ref[0] + p_ref[1]
    inv = 1.0 / jnp.maximum(p[:, 4:5], 1.0)
    mean = p[:, :4] * inv
    h = jnp.maximum(mean @ wl_ref[...] + bl_ref[...] + x_ref[...] @ wr_ref[...], 0.0)
    h_ref[...] = jnp.concatenate([h, jnp.zeros((_BLK, 6), jnp.float32)], axis=1)
    inv_ref[...] = inv


def _l2_body(p_ref, x_ref, inv_ref, wl_ref, bl_ref, wr_ref, h_ref):
    p = p_ref[0] + p_ref[1]
    mean = p[:, :10] * inv_ref[...]
    x10 = x_ref[...][:, :10]
    h = jnp.maximum(mean @ wl_ref[...] + bl_ref[...] + x10 @ wr_ref[...], 0.0)
    z6 = jnp.zeros((_BLK, 6), jnp.float32)
    h_ref[...] = jnp.stack(
        [jnp.concatenate([h[:, :10], z6], axis=1),
         jnp.concatenate([h[:, 10:], z6], axis=1)],
        axis=0,
    )


def _l3_body(p_ref, x_ref, inv_ref, wl_ref, bl_ref, wr_ref, wc_ref, bc_ref, o_ref):
    p = p_ref[...]
    mean = jnp.concatenate([p[0, :, :10], p[1, :, :10]], axis=1) * inv_ref[...]
    x20 = jnp.concatenate([x_ref[0, :, :10], x_ref[1, :, :10]], axis=1)
    h = jnp.maximum(mean @ wl_ref[...] + bl_ref[...] + x20 @ wr_ref[...], 0.0)
    o_ref[...] = h @ wc_ref[...] + bc_ref[...]


def _whole(shape):
    return pl.BlockSpec(shape, lambda i: (0,) * len(shape))


def _rows(d):
    return pl.BlockSpec((_BLK, d), lambda i: (i, 0))


def _part(dp):
    return pl.BlockSpec((2, _BLK, dp), lambda i: (0, i, 0))


def _dense1(part1, x, wl_t, bl, wr_t):
    return pl.pallas_call(
        _l1_body,
        grid=(_GRID,),
        in_specs=[_part(8), _rows(4), _whole((4, 10)), _whole((10,)), _whole((4, 10))],
        out_specs=[_rows(16), _rows(1)],
        out_shape=[
            jax.ShapeDtypeStruct((N_NODES, 16), jnp.float32),
            jax.ShapeDtypeStruct((N_NODES, 1), jnp.float32),
        ],
    )(part1, x, wl_t, bl, wr_t)


def _dense2(part2, h1p, inv, wl_t, bl, wr_t):
    return pl.pallas_call(
        _l2_body,
        grid=(_GRID,),
        in_specs=[
            _part(16), _rows(16), _rows(1),
            _whole((10, 20)), _whole((20,)), _whole((10, 20)),
        ],
        out_specs=pl.BlockSpec((2, _BLK, 16), lambda i: (0, i, 0)),
        out_shape=jax.ShapeDtypeStruct((2, N_NODES, 16), jnp.float32),
    )(part2, h1p, inv, wl_t, bl, wr_t)


def _dense3(part3, h2s, inv, wl_t, bl, wr_t, wc_t, bc):
    return pl.pallas_call(
        _l3_body,
        grid=(_GRID,),
        in_specs=[
            _part(16), pl.BlockSpec((2, _BLK, 16), lambda i: (0, i, 0)), _rows(1),
            _whole((20, 20)), _whole((20,)), _whole((20, 20)),
            _whole((20, 8)), _whole((8,)),
        ],
        out_specs=_rows(8),
        out_shape=jax.ShapeDtypeStruct((N_NODES, 8), jnp.float32),
    )(part3, h2s, inv, wl_t, bl, wr_t, wc_t, bc)


def kernel(x, edge_index, Wl1, bl1, Wr1, Wl2, bl2, Wr2, Wl3, bl3, Wr3, Wc, bc):
    edges3 = edge_index.reshape(2, N_CHUNKS, CH)
    table1 = jnp.concatenate(
        [x, jnp.ones((N_NODES, 1), jnp.float32), jnp.zeros((N_NODES, 3), jnp.float32)],
        axis=1,
    )
    z8 = jnp.zeros((N_PAD, 8), jnp.float32)
    z16 = jnp.zeros((N_PAD, 16), jnp.float32)

    part1 = _agg8(table1, edges3, z8)
    h1p, inv = _dense1(part1, x, Wl1.T, bl1, Wr1.T)

    part2 = _agg16(h1p, edges3, z16)
    h2s = _dense2(part2, h1p, inv, Wl2.T, bl2, Wr2.T)

    part3 = _agg3(h2s, edges3, z16)
    return _dense3(part3, h2s, inv, Wl3.T, bl3, Wr3.T, Wc.T, bc)


# speedup vs baseline: 1.2637x; 1.2637x over previous
import functools

import jax
import jax.numpy as jnp
from jax import lax
from jax.experimental import pallas as pl
from jax.experimental.pallas import tpu as pltpu, tpu_sc as plsc

N_NODES = 100000
N_EDGES = 6400000
NC = 2
NS = 16
NW = NC * NS
CH = 128
N_CHUNKS = N_EDGES // CH
BASE_CHUNKS = N_CHUNKS // NW
REM_CHUNKS = N_CHUNKS % NW
N_PAD = 100096
ROWS_PER_TILE = N_PAD // NS


RING = 4


def _make_agg_generic(dp, split, kblk, ring_r=RING, ring_i=RING):
    mesh = plsc.VectorSubcoreMesh(
        core_axis_name="c", subcore_axis_name="s", num_cores=NC, num_subcores=NS
    )
    stride = NS if split else NW
    nb_total = N_CHUNKS // kblk
    assert nb_total * kblk == N_CHUNKS
    base_blocks = nb_total // stride
    rem_blocks = nb_total % stride
    table_shape = (
        jax.ShapeDtypeStruct((NC, N_NODES, dp), jnp.float32)
        if split else jax.ShapeDtypeStruct((N_NODES, dp), jnp.float32)
    )
    del table_shape

    @functools.partial(
        pl.kernel,
        out_type=jax.ShapeDtypeStruct((NC, N_PAD, dp), jnp.float32),
        mesh=mesh,
        scratch_types=[
            pltpu.VMEM((ring_i, kblk, CH), jnp.int32),
            pltpu.VMEM((ring_i, kblk, CH), jnp.int32),
            pltpu.VMEM((ring_r, kblk, CH, dp), jnp.float32),
            pltpu.VMEM_SHARED((N_PAD, dp), jnp.float32),
            pltpu.SemaphoreType.DMA((ring_i,)),
            pltpu.SemaphoreType.DMA((ring_r,)),
            pltpu.SemaphoreType.DMA((ring_r,)),
        ],
        compiler_params=pltpu.CompilerParams(use_tc_tiling_on_sc=False),
    )
    def agg_kernel(table, edges, zeros, out, src_v, dst_v, rows_v, acc,
                   sem_i, sem_g, sem_s):
        cid = lax.axis_index("c")
        sid = lax.axis_index("s")
        wid = sid if split else sid * NC + cid

        r0 = sid * ROWS_PER_TILE
        pltpu.sync_copy(
            zeros.at[pl.ds(r0, ROWS_PER_TILE)], acc.at[pl.ds(r0, ROWS_PER_TILE)]
        )
        plsc.subcore_barrier()

        n_blocks = jnp.where(wid < rem_blocks, base_blocks + 1, base_blocks)

        def tbl(k_idx):
            return (table.at[cid] if split else table).at[k_idx]

        def fire_idx(m):
            sl = lax.rem(m, ring_i)
            c0 = (wid + m * stride) * kblk
            pltpu.async_copy(edges.at[0, pl.ds(c0, kblk)], src_v.at[sl],
                             sem_i.at[sl])
            pltpu.async_copy(edges.at[1, pl.ds(c0, kblk)], dst_v.at[sl],
                             sem_i.at[sl])

        def drain_idx(m):
            sl = lax.rem(m, ring_i)
            c0 = (wid + m * stride) * kblk
            pltpu.make_async_copy(edges.at[0, pl.ds(c0, kblk)], src_v.at[sl],
                                  sem_i.at[sl]).wait()
            pltpu.make_async_copy(edges.at[1, pl.ds(c0, kblk)], dst_v.at[sl],
                                  sem_i.at[sl]).wait()

        def fire_gath(m):
            sl = lax.rem(m, ring_r)
            si = lax.rem(m, ring_i)
            for k in range(kblk):
                pltpu.async_copy(tbl(src_v.at[si, k]), rows_v.at[sl, k],
                                 sem_g.at[sl])

        def drain_gath(m):
            sl = lax.rem(m, ring_r)
            si = lax.rem(m, ring_i)
            for k in range(kblk):
                pltpu.make_async_copy(tbl(src_v.at[si, k]), rows_v.at[sl, k],
                                      sem_g.at[sl]).wait()

        def fire_scat(m):
            sl = lax.rem(m, ring_r)
            si = lax.rem(m, ring_i)
            for k in range(kblk):
                pltpu.async_copy(rows_v.at[sl, k], acc.at[dst_v.at[si, k]],
                                 sem_s.at[sl], add=True)

        def drain_scat(m):
            sl = lax.rem(m, ring_r)
            si = lax.rem(m, ring_i)
            for k in range(kblk):
                pltpu.make_async_copy(rows_v.at[sl, k], acc.at[dst_v.at[si, k]],
                                      sem_s.at[sl]).wait()

        fire_idx(0)
        fire_idx(1)
        drain_idx(0)
        fire_gath(0)

        @pl.loop(0, n_blocks)
        def _(m):
            drain_gath(m)
            fire_scat(m)

            @pl.when(m >= 2)
            def _():
                drain_scat(m - 2)

            @pl.when(m + 1 < n_blocks)
            def _():
                drain_idx(m + 1)
                fire_gath(m + 1)

            @pl.when(m + 2 < n_blocks)
            def _():
                fire_idx(m + 2)

        drain_scat(n_blocks - 2)
        drain_scat(n_blocks - 1)

        plsc.subcore_barrier()
        pltpu.sync_copy(
            acc.at[pl.ds(r0, ROWS_PER_TILE)], out.at[cid, pl.ds(r0, ROWS_PER_TILE)]
        )

    return agg_kernel


def _make_agg(dp, kblk):
    ring_r = 4 if dp == 8 else 3
    return _make_agg_generic(dp, split=False, kblk=kblk, ring_r=ring_r)


def _make_agg3():
    return _make_agg_generic(16, split=True, kblk=4, ring_r=3)


_agg8 = _make_agg(8, kblk=8)
_agg16 = _make_agg(16, kblk=4)
_agg3 = _make_agg3()

_BLK = 2000
_GRID = N_NODES // _BLK


def _l1_body(p_ref, x_ref, wl_ref, bl_ref, wr_ref, h_ref, inv_ref):
    p = p_ref[0] + p_ref[1]
    inv = 1.0 / jnp.maximum(p[:, 4:5], 1.0)
    mean = p[:, :4] * inv
    h = jnp.maximum(mean @ wl_ref[...] + bl_ref[...] + x_ref[...] @ wr_ref[...], 0.0)
    h_ref[...] = jnp.concatenate([h, jnp.zeros((_BLK, 6), jnp.float32)], axis=1)
    inv_ref[...] = inv


def _l2_body(p_ref, x_ref, inv_ref, wl_ref, bl_ref, wr_ref, h_ref):
    p = p_ref[0] + p_ref[1]
    mean = p[:, :10] * inv_ref[...]
    x10 = x_ref[...][:, :10]
    h = jnp.maximum(mean @ wl_ref[...] + bl_ref[...] + x10 @ wr_ref[...], 0.0)
    z6 = jnp.zeros((_BLK, 6), jnp.float32)
    h_ref[...] = jnp.stack(
        [jnp.concatenate([h[:, :10], z6], axis=1),
         jnp.concatenate([h[:, 10:], z6], axis=1)],
        axis=0,
    )


def _l3_body(p_ref, x_ref, inv_ref, wl_ref, bl_ref, wr_ref, wc_ref, bc_ref, o_ref):
    p = p_ref[...]
    mean = jnp.concatenate([p[0, :, :10], p[1, :, :10]], axis=1) * inv_ref[...]
    x20 = jnp.concatenate([x_ref[0, :, :10], x_ref[1, :, :10]], axis=1)
    h = jnp.maximum(mean @ wl_ref[...] + bl_ref[...] + x20 @ wr_ref[...], 0.0)
    o_ref[...] = h @ wc_ref[...] + bc_ref[...]


def _whole(shape):
    return pl.BlockSpec(shape, lambda i: (0,) * len(shape))


def _rows(d):
    return pl.BlockSpec((_BLK, d), lambda i: (i, 0))


def _part(dp):
    return pl.BlockSpec((2, _BLK, dp), lambda i: (0, i, 0))


def _dense1(part1, x, wl_t, bl, wr_t):
    return pl.pallas_call(
        _l1_body,
        grid=(_GRID,),
        in_specs=[_part(8), _rows(4), _whole((4, 10)), _whole((10,)), _whole((4, 10))],
        out_specs=[_rows(16), _rows(1)],
        out_shape=[
            jax.ShapeDtypeStruct((N_NODES, 16), jnp.float32),
            jax.ShapeDtypeStruct((N_NODES, 1), jnp.float32),
        ],
    )(part1, x, wl_t, bl, wr_t)


def _dense2(part2, h1p, inv, wl_t, bl, wr_t):
    return pl.pallas_call(
        _l2_body,
        grid=(_GRID,),
        in_specs=[
            _part(16), _rows(16), _rows(1),
            _whole((10, 20)), _whole((20,)), _whole((10, 20)),
        ],
        out_specs=pl.BlockSpec((2, _BLK, 16), lambda i: (0, i, 0)),
        out_shape=jax.ShapeDtypeStruct((2, N_NODES, 16), jnp.float32),
    )(part2, h1p, inv, wl_t, bl, wr_t)


def _dense3(part3, h2s, inv, wl_t, bl, wr_t, wc_t, bc):
    return pl.pallas_call(
        _l3_body,
        grid=(_GRID,),
        in_specs=[
            _part(16), pl.BlockSpec((2, _BLK, 16), lambda i: (0, i, 0)), _rows(1),
            _whole((20, 20)), _whole((20,)), _whole((20, 20)),
            _whole((20, 8)), _whole((8,)),
        ],
        out_specs=_rows(8),
        out_shape=jax.ShapeDtypeStruct((N_NODES, 8), jnp.float32),
    )(part3, h2s, inv, wl_t, bl, wr_t, wc_t, bc)


def kernel(x, edge_index, Wl1, bl1, Wr1, Wl2, bl2, Wr2, Wl3, bl3, Wr3, Wc, bc):
    edges3 = edge_index.reshape(2, N_CHUNKS, CH)
    table1 = jnp.concatenate(
        [x, jnp.ones((N_NODES, 1), jnp.float32), jnp.zeros((N_NODES, 3), jnp.float32)],
        axis=1,
    )
    z8 = jnp.zeros((N_PAD, 8), jnp.float32)
    z16 = jnp.zeros((N_PAD, 16), jnp.float32)

    part1 = _agg8(table1, edges3, z8)
    h1p, inv = _dense1(part1, x, Wl1.T, bl1, Wr1.T)

    part2 = _agg16(h1p, edges3, z16)
    h2s = _dense2(part2, h1p, inv, Wl2.T, bl2, Wr2.T)

    part3 = _agg3(h2s, edges3, z16)
    return _dense3(part3, h2s, inv, Wl3.T, bl3, Wr3.T, Wc.T, bc)


# speedup vs baseline: 1.2962x; 1.0258x over previous
import functools

import jax
import jax.numpy as jnp
from jax import lax
from jax.experimental import pallas as pl
from jax.experimental.pallas import tpu as pltpu, tpu_sc as plsc

N_NODES = 100000
N_EDGES = 6400000
NC = 2
NS = 16
NW = NC * NS
CH = 128
N_CHUNKS = N_EDGES // CH
BASE_CHUNKS = N_CHUNKS // NW
REM_CHUNKS = N_CHUNKS % NW
N_PAD = 100096
ROWS_PER_TILE = N_PAD // NS


RING = 4


def _make_agg_generic(dp, split, kblk, ring_r=RING, ring_i=RING):
    mesh = plsc.VectorSubcoreMesh(
        core_axis_name="c", subcore_axis_name="s", num_cores=NC, num_subcores=NS
    )
    stride = NS if split else NW
    nb_total = N_CHUNKS // kblk
    assert nb_total * kblk == N_CHUNKS
    base_blocks = nb_total // stride
    rem_blocks = nb_total % stride
    table_shape = (
        jax.ShapeDtypeStruct((NC, N_NODES, dp), jnp.float32)
        if split else jax.ShapeDtypeStruct((N_NODES, dp), jnp.float32)
    )
    del table_shape

    @functools.partial(
        pl.kernel,
        out_type=jax.ShapeDtypeStruct((NC, N_PAD, dp), jnp.float32),
        mesh=mesh,
        scratch_types=[
            pltpu.VMEM((ring_i, kblk, 2, CH), jnp.int32),
            pltpu.VMEM((ring_r, kblk, CH, dp), jnp.float32),
            pltpu.VMEM_SHARED((N_PAD, dp), jnp.float32),
            pltpu.SemaphoreType.DMA((ring_i,)),
            pltpu.SemaphoreType.DMA((ring_r,)),
            pltpu.SemaphoreType.DMA((ring_r,)),
        ],
        compiler_params=pltpu.CompilerParams(use_tc_tiling_on_sc=False),
    )
    def agg_kernel(table, edges, zeros, out, idx_v, rows_v, acc,
                   sem_i, sem_g, sem_s):
        cid = lax.axis_index("c")
        sid = lax.axis_index("s")
        wid = sid if split else sid * NC + cid

        r0 = sid * ROWS_PER_TILE
        pltpu.sync_copy(
            zeros.at[pl.ds(r0, ROWS_PER_TILE)], acc.at[pl.ds(r0, ROWS_PER_TILE)]
        )
        plsc.subcore_barrier()

        n_blocks = jnp.where(wid < rem_blocks, base_blocks + 1, base_blocks)

        def tbl(k_idx):
            return (table.at[cid] if split else table).at[k_idx]

        def fire_idx(m):
            sl = lax.rem(m, ring_i)
            c0 = (wid + m * stride) * kblk
            pltpu.async_copy(edges.at[pl.ds(c0, kblk)], idx_v.at[sl],
                             sem_i.at[sl])

        def drain_idx(m):
            sl = lax.rem(m, ring_i)
            c0 = (wid + m * stride) * kblk
            pltpu.make_async_copy(edges.at[pl.ds(c0, kblk)], idx_v.at[sl],
                                  sem_i.at[sl]).wait()

        def fire_gath(m):
            sl = lax.rem(m, ring_r)
            si = lax.rem(m, ring_i)
            for k in range(kblk):
                pltpu.async_copy(tbl(idx_v.at[si, k, 0]), rows_v.at[sl, k],
                                 sem_g.at[sl])

        def drain_gath(m):
            sl = lax.rem(m, ring_r)
            si = lax.rem(m, ring_i)
            for k in range(kblk):
                pltpu.make_async_copy(tbl(idx_v.at[si, k, 0]), rows_v.at[sl, k],
                                      sem_g.at[sl]).wait()

        def fire_scat(m):
            sl = lax.rem(m, ring_r)
            si = lax.rem(m, ring_i)
            for k in range(kblk):
                pltpu.async_copy(rows_v.at[sl, k], acc.at[idx_v.at[si, k, 1]],
                                 sem_s.at[sl], add=True)

        def drain_scat(m):
            sl = lax.rem(m, ring_r)
            si = lax.rem(m, ring_i)
            for k in range(kblk):
                pltpu.make_async_copy(rows_v.at[sl, k], acc.at[idx_v.at[si, k, 1]],
                                      sem_s.at[sl]).wait()

        fire_idx(0)
        fire_idx(1)
        drain_idx(0)
        fire_gath(0)

        @pl.loop(0, n_blocks)
        def _(m):
            drain_gath(m)
            fire_scat(m)

            @pl.when(m >= 2)
            def _():
                drain_scat(m - 2)

            @pl.when(m + 1 < n_blocks)
            def _():
                drain_idx(m + 1)
                fire_gath(m + 1)

            @pl.when(m + 2 < n_blocks)
            def _():
                fire_idx(m + 2)

        drain_scat(n_blocks - 2)
        drain_scat(n_blocks - 1)

        plsc.subcore_barrier()
        pltpu.sync_copy(
            acc.at[pl.ds(r0, ROWS_PER_TILE)], out.at[cid, pl.ds(r0, ROWS_PER_TILE)]
        )

    return agg_kernel


def _make_agg(dp, kblk):
    ring_r = 4 if dp == 8 else 3
    return _make_agg_generic(dp, split=False, kblk=kblk, ring_r=ring_r)


def _make_agg3():
    return _make_agg_generic(16, split=True, kblk=4, ring_r=3)


_agg8 = _make_agg(8, kblk=10)
_agg16 = _make_agg(16, kblk=4)
_agg3 = _make_agg3()

_BLK = 2000
_GRID = N_NODES // _BLK


def _l1_body(p_ref, x_ref, wl_ref, bl_ref, wr_ref, h_ref, inv_ref):
    p = p_ref[0] + p_ref[1]
    inv = 1.0 / jnp.maximum(p[:, 4:5], 1.0)
    mean = p[:, :4] * inv
    h = jnp.maximum(mean @ wl_ref[...] + bl_ref[...] + x_ref[...] @ wr_ref[...], 0.0)
    h_ref[...] = jnp.concatenate([h, jnp.zeros((_BLK, 6), jnp.float32)], axis=1)
    inv_ref[...] = inv


def _l2_body(p_ref, x_ref, inv_ref, wl_ref, bl_ref, wr_ref, h_ref):
    p = p_ref[0] + p_ref[1]
    mean = p[:, :10] * inv_ref[...]
    x10 = x_ref[...][:, :10]
    h = jnp.maximum(mean @ wl_ref[...] + bl_ref[...] + x10 @ wr_ref[...], 0.0)
    z6 = jnp.zeros((_BLK, 6), jnp.float32)
    h_ref[...] = jnp.stack(
        [jnp.concatenate([h[:, :10], z6], axis=1),
         jnp.concatenate([h[:, 10:], z6], axis=1)],
        axis=0,
    )


def _l3_body(p_ref, x_ref, inv_ref, wl_ref, bl_ref, wr_ref, wc_ref, bc_ref, o_ref):
    p = p_ref[...]
    mean = jnp.concatenate([p[0, :, :10], p[1, :, :10]], axis=1) * inv_ref[...]
    x20 = jnp.concatenate([x_ref[0, :, :10], x_ref[1, :, :10]], axis=1)
    h = jnp.maximum(mean @ wl_ref[...] + bl_ref[...] + x20 @ wr_ref[...], 0.0)
    o_ref[...] = h @ wc_ref[...] + bc_ref[...]


def _whole(shape):
    return pl.BlockSpec(shape, lambda i: (0,) * len(shape))


def _rows(d):
    return pl.BlockSpec((_BLK, d), lambda i: (i, 0))


def _part(dp):
    return pl.BlockSpec((2, _BLK, dp), lambda i: (0, i, 0))


def _dense1(part1, x, wl_t, bl, wr_t):
    return pl.pallas_call(
        _l1_body,
        grid=(_GRID,),
        in_specs=[_part(8), _rows(4), _whole((4, 10)), _whole((10,)), _whole((4, 10))],
        out_specs=[_rows(16), _rows(1)],
        out_shape=[
            jax.ShapeDtypeStruct((N_NODES, 16), jnp.float32),
            jax.ShapeDtypeStruct((N_NODES, 1), jnp.float32),
        ],
    )(part1, x, wl_t, bl, wr_t)


def _dense2(part2, h1p, inv, wl_t, bl, wr_t):
    return pl.pallas_call(
        _l2_body,
        grid=(_GRID,),
        in_specs=[
            _part(16), _rows(16), _rows(1),
            _whole((10, 20)), _whole((20,)), _whole((10, 20)),
        ],
        out_specs=pl.BlockSpec((2, _BLK, 16), lambda i: (0, i, 0)),
        out_shape=jax.ShapeDtypeStruct((2, N_NODES, 16), jnp.float32),
    )(part2, h1p, inv, wl_t, bl, wr_t)


def _dense3(part3, h2s, inv, wl_t, bl, wr_t, wc_t, bc):
    return pl.pallas_call(
        _l3_body,
        grid=(_GRID,),
        in_specs=[
            _part(16), pl.BlockSpec((2, _BLK, 16), lambda i: (0, i, 0)), _rows(1),
            _whole((20, 20)), _whole((20,)), _whole((20, 20)),
            _whole((20, 8)), _whole((8,)),
        ],
        out_specs=_rows(8),
        out_shape=jax.ShapeDtypeStruct((N_NODES, 8), jnp.float32),
    )(part3, h2s, inv, wl_t, bl, wr_t, wc_t, bc)


def kernel(x, edge_index, Wl1, bl1, Wr1, Wl2, bl2, Wr2, Wl3, bl3, Wr3, Wc, bc):
    edges3 = jnp.transpose(edge_index.reshape(2, N_CHUNKS, CH), (1, 0, 2))
    table1 = jnp.concatenate(
        [x, jnp.ones((N_NODES, 1), jnp.float32), jnp.zeros((N_NODES, 3), jnp.float32)],
        axis=1,
    )
    z8 = jnp.zeros((N_PAD, 8), jnp.float32)
    z16 = jnp.zeros((N_PAD, 16), jnp.float32)

    part1 = _agg8(table1, edges3, z8)
    h1p, inv = _dense1(part1, x, Wl1.T, bl1, Wr1.T)

    part2 = _agg16(h1p, edges3, z16)
    h2s = _dense2(part2, h1p, inv, Wl2.T, bl2, Wr2.T)

    part3 = _agg3(h2s, edges3, z16)
    return _dense3(part3, h2s, inv, Wl3.T, bl3, Wr3.T, Wc.T, bc)
